# Initial kernel scaffold; baseline (speedup 1.0000x reference)
import jax
import jax.numpy as jnp
from jax.experimental import pallas as pl


def kernel(lang, frames, actions, lens_lang, lens_frames, pos, pe):
    raise NotImplementedError("write your pallas kernel here")


# speedup vs baseline: 1.3978x; 1.3978x over previous
import functools

import jax
import jax.numpy as jnp
from jax import lax
from jax.experimental import pallas as pl
from jax.experimental.pallas import tpu as pltpu
from jax.experimental.pallas import tpu_sc as plsc

D = 1024
B = 16
L = 1024
N_ROWS = B * L
NW = 32
W_ROWS = N_ROWS // NW
CH = 16
NCH = W_ROWS // CH
NV = D // 16
SCALE = 0.03125


def _body(lang_hbm, frames_hbm, actions_hbm, lens_hbm, pos_lang_hbm,
          pos0_hbm, pe_hbm, lang_out, frames_out, actions_out,
          idx_buf, in_buf, act_buf, pe_buf, pos0_buf, lens_buf,
          sem_pe, sem_in, sem_act):
    c = lax.axis_index("c")
    s = lax.axis_index("s")
    wid = s * 2 + c
    iota = lax.iota(jnp.int32, 16)

    def lang_chunk(k, carry):
        row0 = wid * W_ROWS + k * CH
        pltpu.sync_copy(pos_lang_hbm.at[pl.ds(row0, CH)], idx_buf)
        cp_pe = pltpu.async_copy(pe_hbm.at[idx_buf], pe_buf, sem_pe)
        cp_in = pltpu.async_copy(lang_hbm.at[pl.ds(row0, CH)], in_buf, sem_in)
        cp_pe.wait()
        cp_in.wait()

        def row(r, _):
            for d in range(NV):
                sl = pl.ds(d * 16, 16)
                plsc.addupdate(in_buf.at[r, sl], pe_buf[r, sl] * SCALE)
            return 0

        lax.fori_loop(0, CH, row, 0)
        pltpu.sync_copy(in_buf, lang_out.at[pl.ds(row0, CH)])
        return carry

    lax.fori_loop(0, NCH, lang_chunk, 0)

    pltpu.sync_copy(pos0_hbm, pos0_buf)
    pltpu.sync_copy(lens_hbm, lens_buf)
    b = wid // 2
    jhalf = (wid % 2) * (L // 2)
    l_vec = plsc.load_gather(lens_buf, [jnp.full((16,), b, jnp.int32)])

    def fa_chunk(k, carry):
        jbase = jhalf + k * CH
        row0 = b * L + jbase
        pe_rows = plsc.load_gather(pos0_buf, [l_vec + (jbase + iota)])
        idx_buf[...] = pe_rows
        cp_pe = pltpu.async_copy(pe_hbm.at[idx_buf], pe_buf, sem_pe)
        cp_f = pltpu.async_copy(frames_hbm.at[pl.ds(row0, CH)], in_buf, sem_in)
        cp_a = pltpu.async_copy(actions_hbm.at[pl.ds(row0, CH)], act_buf, sem_act)
        cp_pe.wait()
        cp_f.wait()
        cp_a.wait()

        def row(r, _):
            for d in range(NV):
                sl = pl.ds(d * 16, 16)
                v = pe_buf[r, sl] * SCALE
                plsc.addupdate(in_buf.at[r, sl], v)
                plsc.addupdate(act_buf.at[r, sl], v)
            return 0

        lax.fori_loop(0, CH, row, 0)
        pltpu.sync_copy(in_buf, frames_out.at[pl.ds(row0, CH)])
        pltpu.sync_copy(act_buf, actions_out.at[pl.ds(row0, CH)])
        return carry

    lax.fori_loop(0, NCH, fa_chunk, 0)


def kernel(lang, frames, actions, lens_lang, lens_frames, pos, pe):
    del lens_frames
    pos = pos.astype(jnp.int32)
    lang2d = lang.reshape(N_ROWS, D)
    frames2d = frames.reshape(N_ROWS, D)
    actions2d = actions.reshape(N_ROWS, D)
    pos_lang = pos[:, :L].reshape(N_ROWS)
    pos0 = pos[0]
    lo, fo, ao = _pallas_run(lang2d, frames2d, actions2d,
                             lens_lang.astype(jnp.int32), pos_lang, pos0, pe)
    return (lo.reshape(B, L, D), fo.reshape(B, L, D), ao.reshape(B, L, D))


def _pallas_run(lang2d, frames2d, actions2d, lens_i32, pos_lang, pos0, pe):
    mesh = plsc.VectorSubcoreMesh(core_axis_name="c", subcore_axis_name="s")
    f32 = jnp.float32
    out_type = (
        jax.ShapeDtypeStruct((N_ROWS, D), f32),
        jax.ShapeDtypeStruct((N_ROWS, D), f32),
        jax.ShapeDtypeStruct((N_ROWS, D), f32),
    )
    return pl.kernel(
        _body,
        out_type,
        mesh=mesh,
        compiler_params=pltpu.CompilerParams(needs_layout_passes=False),
        scratch_types=[
            pltpu.VMEM((CH,), jnp.int32),
            pltpu.VMEM((CH, D), f32),
            pltpu.VMEM((CH, D), f32),
            pltpu.VMEM((CH, D), f32),
            pltpu.VMEM((2 * L,), jnp.int32),
            pltpu.VMEM((B,), jnp.int32),
            pltpu.SemaphoreType.DMA,
            pltpu.SemaphoreType.DMA,
            pltpu.SemaphoreType.DMA,
        ],
    )(lang2d, frames2d, actions2d, lens_i32, pos_lang, pos0, pe)


# speedup vs baseline: 1.9777x; 1.4149x over previous
import jax
import jax.numpy as jnp
from jax import lax
from jax.experimental import pallas as pl
from jax.experimental.pallas import tpu as pltpu
from jax.experimental.pallas import tpu_sc as plsc

D = 1024
B = 16
L = 1024
N_ROWS = B * L
NW = 32
W_ROWS = N_ROWS // NW
CH = 16
NCH = W_ROWS // CH
NV = D // 16
NBUF = 2
SCALE = 0.03125


def _body(lang_hbm, frames_hbm, actions_hbm, lens_hbm, pos_lang_hbm,
          pos0_hbm, pe_hbm, lang_out, frames_out, actions_out,
          idx_all, in_buf, act_buf, pe_buf, pos0_buf, lens_buf,
          sem_pe0, sem_pe1, sem_ld0, sem_ld1, sem_st0, sem_st1):
    c = lax.axis_index("c")
    s = lax.axis_index("s")
    wid = s * 2 + c
    iota = lax.iota(jnp.int32, 16)
    base = wid * W_ROWS
    sem_pe = [sem_pe0, sem_pe1]
    sem_ld = [sem_ld0, sem_ld1]
    sem_st = [sem_st0, sem_st1]

    def idx_ref(k):
        return idx_all.at[pl.ds(k * CH, CH)]

    def run_phase(start, wait_loads, compute, start_store, wait_store):
        start(0, 0)

        def step(k, p):
            knext = k + 1

            pnext = (p + 1) % NBUF

            @pl.when(knext < NCH)
            def _():
                @pl.when(knext >= NBUF)
                def _():
                    wait_store(knext - NBUF, pnext)
                start(knext, pnext)

            wait_loads(k, p)
            compute(p)
            start_store(k, p)

        def outer(i, carry):
            kk = i * NBUF
            for p in range(NBUF):
                step(kk + p, p)
            return carry

        lax.fori_loop(0, NCH // NBUF, outer, 0)
        wait_store(NCH - 2, 0)
        wait_store(NCH - 1, 1)

    pltpu.sync_copy(pos_lang_hbm.at[pl.ds(base, W_ROWS)], idx_all)

    def startL(k, p):
        row0 = base + k * CH
        pltpu.async_copy(pe_hbm.at[idx_ref(k)], pe_buf.at[p], sem_pe[p])
        pltpu.async_copy(lang_hbm.at[pl.ds(row0, CH)], in_buf.at[p], sem_ld[p])

    def wait_loadsL(k, p):
        row0 = base + k * CH
        pltpu.make_async_copy(pe_hbm.at[idx_ref(k)], pe_buf.at[p], sem_pe[p]).wait()
        pltpu.make_async_copy(lang_hbm.at[pl.ds(row0, CH)], in_buf.at[p], sem_ld[p]).wait()

    def computeL(p):
        def row(r, carry):
            for d in range(NV):
                sl = pl.ds(d * 16, 16)
                plsc.addupdate(in_buf.at[p, r, sl], pe_buf[p, r, sl] * SCALE)
            return carry

        lax.fori_loop(0, CH, row, 0)

    def start_storeL(k, p):
        row0 = base + k * CH
        pltpu.async_copy(in_buf.at[p], lang_out.at[pl.ds(row0, CH)], sem_st[p])

    def wait_storeL(k, p):
        row0 = base + k * CH
        pltpu.make_async_copy(in_buf.at[p], lang_out.at[pl.ds(row0, CH)], sem_st[p]).wait()

    run_phase(startL, wait_loadsL, computeL, start_storeL, wait_storeL)

    pltpu.sync_copy(pos0_hbm, pos0_buf)
    pltpu.sync_copy(lens_hbm, lens_buf)
    b = wid // 2
    jhalf = (wid % 2) * (L // 2)
    l_vec = plsc.load_gather(lens_buf, [jnp.full((16,), b, jnp.int32)])
    rbase = b * L + jhalf

    def build_idx(k, carry):
        pe_rows = plsc.load_gather(pos0_buf, [l_vec + (jhalf + k * CH + iota)])
        idx_all[pl.ds(k * CH, CH)] = pe_rows
        return carry

    lax.fori_loop(0, NCH, build_idx, 0)

    def startF(k, p):
        row0 = rbase + k * CH
        pltpu.async_copy(pe_hbm.at[idx_ref(k)], pe_buf.at[p], sem_pe[p])
        pltpu.async_copy(frames_hbm.at[pl.ds(row0, CH)], in_buf.at[p], sem_ld[p])
        pltpu.async_copy(actions_hbm.at[pl.ds(row0, CH)], act_buf.at[p], sem_ld[p])

    def wait_loadsF(k, p):
        row0 = rbase + k * CH
        pltpu.make_async_copy(pe_hbm.at[idx_ref(k)], pe_buf.at[p], sem_pe[p]).wait()
        pltpu.make_async_copy(frames_hbm.at[pl.ds(row0, CH)], in_buf.at[p], sem_ld[p]).wait()
        pltpu.make_async_copy(actions_hbm.at[pl.ds(row0, CH)], act_buf.at[p], sem_ld[p]).wait()

    def computeF(p):
        def row(r, carry):
            for d in range(NV):
                sl = pl.ds(d * 16, 16)
                v = pe_buf[p, r, sl] * SCALE
                plsc.addupdate(in_buf.at[p, r, sl], v)
                plsc.addupdate(act_buf.at[p, r, sl], v)
            return carry

        lax.fori_loop(0, CH, row, 0)

    def start_storeF(k, p):
        row0 = rbase + k * CH
        pltpu.async_copy(in_buf.at[p], frames_out.at[pl.ds(row0, CH)], sem_st[p])
        pltpu.async_copy(act_buf.at[p], actions_out.at[pl.ds(row0, CH)], sem_st[p])

    def wait_storeF(k, p):
        row0 = rbase + k * CH
        pltpu.make_async_copy(in_buf.at[p], frames_out.at[pl.ds(row0, CH)], sem_st[p]).wait()
        pltpu.make_async_copy(act_buf.at[p], actions_out.at[pl.ds(row0, CH)], sem_st[p]).wait()

    run_phase(startF, wait_loadsF, computeF, start_storeF, wait_storeF)


def kernel(lang, frames, actions, lens_lang, lens_frames, pos, pe):
    del lens_frames
    pos = pos.astype(jnp.int32)
    lang2d = lang.reshape(N_ROWS, D)
    frames2d = frames.reshape(N_ROWS, D)
    actions2d = actions.reshape(N_ROWS, D)
    pos_lang = pos[:, :L].reshape(N_ROWS)
    pos0 = pos[0]
    lo, fo, ao = _pallas_run(lang2d, frames2d, actions2d,
                             lens_lang.astype(jnp.int32), pos_lang, pos0, pe)
    return (lo.reshape(B, L, D), fo.reshape(B, L, D), ao.reshape(B, L, D))


def _pallas_run(lang2d, frames2d, actions2d, lens_i32, pos_lang, pos0, pe):
    mesh = plsc.VectorSubcoreMesh(core_axis_name="c", subcore_axis_name="s")
    f32 = jnp.float32
    out_type = (
        jax.ShapeDtypeStruct((N_ROWS, D), f32),
        jax.ShapeDtypeStruct((N_ROWS, D), f32),
        jax.ShapeDtypeStruct((N_ROWS, D), f32),
    )
    return pl.kernel(
        _body,
        out_type,
        mesh=mesh,
        compiler_params=pltpu.CompilerParams(needs_layout_passes=False),
        scratch_types=[
            pltpu.VMEM((W_ROWS,), jnp.int32),
            pltpu.VMEM((NBUF, CH, D), f32),
            pltpu.VMEM((NBUF, CH, D), f32),
            pltpu.VMEM((NBUF, CH, D), f32),
            pltpu.VMEM((2 * L,), jnp.int32),
            pltpu.VMEM((B,), jnp.int32),
            pltpu.SemaphoreType.DMA,
            pltpu.SemaphoreType.DMA,
            pltpu.SemaphoreType.DMA,
            pltpu.SemaphoreType.DMA,
            pltpu.SemaphoreType.DMA,
            pltpu.SemaphoreType.DMA,
        ],
    )(lang2d, frames2d, actions2d, lens_i32, pos_lang, pos0, pe)
